# baseline (speedup 1.0000x reference)
import jax
import jax.numpy as jnp
from jax.experimental import pallas as pl
from jax.experimental.pallas import tpu as pltpu

_TOKENS = 16384
_DIM = 2048
_EXPERTS = 64
_CHUNK = 256
_NCHUNKS = _TOKENS // _CHUNK
_NBUF = 8


def _router_body(x_hbm, w_ref, b_ref, out_ref, buf, sem):
    def chunk_copy(chunk, slot):
        return pltpu.make_async_copy(
            x_hbm.at[pl.ds(chunk * _CHUNK, _CHUNK), :],
            buf.at[slot],
            sem.at[slot],
        )

    for s in range(_NBUF):
        chunk_copy(s, s).start()

    def step(i, carry):
        slot = jax.lax.rem(i, _NBUF)
        chunk_copy(i, slot).wait()
        out_ref[pl.ds(i * _CHUNK, _CHUNK), :] = jax.lax.dot_general(
            buf[slot],
            w_ref[...],
            dimension_numbers=(((1,), (1,)), ((), ())),
            preferred_element_type=jnp.float32,
        ) + b_ref[...]

        nxt = i + _NBUF

        @pl.when(nxt < _NCHUNKS)
        def _():
            chunk_copy(nxt, slot).start()

        return carry

    jax.lax.fori_loop(0, _NCHUNKS, step, 0, unroll=_NBUF)


@jax.jit
def kernel(x, W, b):
    out = pl.pallas_call(
        _router_body,
        in_specs=[
            pl.BlockSpec(memory_space=pltpu.MemorySpace.HBM),
            pl.BlockSpec(memory_space=pltpu.VMEM),
            pl.BlockSpec(memory_space=pltpu.VMEM),
        ],
        out_specs=pl.BlockSpec(memory_space=pltpu.VMEM),
        out_shape=jax.ShapeDtypeStruct((_TOKENS, _EXPERTS), jnp.float32),
        scratch_shapes=[
            pltpu.VMEM((_NBUF, _CHUNK, _DIM), jnp.float32),
            pltpu.SemaphoreType.DMA((_NBUF,)),
        ],
    )(x, W, b.reshape(1, _EXPERTS))
    return out


# speedup vs baseline: 1.0577x; 1.0577x over previous
import jax
import jax.numpy as jnp
from jax.experimental import pallas as pl
from jax.experimental.pallas import tpu as pltpu

_TOKENS = 16384
_DIM = 2048
_EXPERTS = 64
_CHUNK = 256
_NCHUNKS = _TOKENS // _CHUNK
_NBUF = 8


def _router_body(x_hbm, w_ref, b_ref, out_ref, buf, sem):
    def chunk_copy(chunk, slot):
        return pltpu.make_async_copy(
            x_hbm.at[pl.ds(chunk * _CHUNK, _CHUNK), :],
            buf.at[slot],
            sem.at[slot],
        )

    for s in range(_NBUF):
        chunk_copy(s, s).start()

    def step(i, carry):
        slot = jax.lax.rem(i, _NBUF)
        chunk_copy(i, slot).wait()
        out_ref[pl.ds(i * _CHUNK, _CHUNK), :] = jax.lax.dot_general(
            buf[slot],
            w_ref[...],
            dimension_numbers=(((1,), (1,)), ((), ())),
            preferred_element_type=jnp.float32,
        ) + b_ref[...]

        nxt = i + _NBUF

        @pl.when(nxt < _NCHUNKS)
        def _():
            chunk_copy(nxt, slot).start()

        return carry

    jax.lax.fori_loop(0, _NCHUNKS, step, 0)


@jax.jit
def kernel(x, W, b):
    out = pl.pallas_call(
        _router_body,
        in_specs=[
            pl.BlockSpec(memory_space=pltpu.MemorySpace.HBM),
            pl.BlockSpec(memory_space=pltpu.VMEM),
            pl.BlockSpec(memory_space=pltpu.VMEM),
        ],
        out_specs=pl.BlockSpec(memory_space=pltpu.VMEM),
        out_shape=jax.ShapeDtypeStruct((_TOKENS, _EXPERTS), jnp.float32),
        scratch_shapes=[
            pltpu.VMEM((_NBUF, _CHUNK, _DIM), jnp.float32),
            pltpu.SemaphoreType.DMA((_NBUF,)),
        ],
    )(x, W, b.reshape(1, _EXPERTS))
    return out


# speedup vs baseline: 1.1090x; 1.0485x over previous
import jax
import jax.numpy as jnp
from jax.experimental import pallas as pl
from jax.experimental.pallas import tpu as pltpu

_TOKENS = 16384
_DIM = 2048
_EXPERTS = 64
_BLOCK_T = 1024
_SPLIT = 2
_SUB = _BLOCK_T // _SPLIT


def _router_body(xa_ref, xb_ref, w_ref, b_ref, out_ref):
    w = w_ref[...]
    b = b_ref[...]
    out_ref[0:_SUB, :] = jax.lax.dot_general(
        xa_ref[...], w,
        dimension_numbers=(((1,), (1,)), ((), ())),
        preferred_element_type=jnp.float32,
    ) + b
    out_ref[_SUB:_BLOCK_T, :] = jax.lax.dot_general(
        xb_ref[...], w,
        dimension_numbers=(((1,), (1,)), ((), ())),
        preferred_element_type=jnp.float32,
    ) + b


@jax.jit
def kernel(x, W, b):
    grid = (_TOKENS // _BLOCK_T,)
    out = pl.pallas_call(
        _router_body,
        grid=grid,
        in_specs=[
            pl.BlockSpec((_SUB, _DIM), lambda i: (2 * i, 0)),
            pl.BlockSpec((_SUB, _DIM), lambda i: (2 * i + 1, 0)),
            pl.BlockSpec((_EXPERTS, _DIM), lambda i: (0, 0)),
            pl.BlockSpec((1, _EXPERTS), lambda i: (0, 0)),
        ],
        out_specs=pl.BlockSpec((_BLOCK_T, _EXPERTS), lambda i: (i, 0)),
        out_shape=jax.ShapeDtypeStruct((_TOKENS, _EXPERTS), jnp.float32),
        compiler_params=pltpu.CompilerParams(
            dimension_semantics=("arbitrary",),
        ),
    )(x, x, W, b.reshape(1, _EXPERTS))
    return out
